# Initial kernel scaffold; baseline (speedup 1.0000x reference)
import jax
import jax.numpy as jnp
from jax.experimental import pallas as pl


def kernel(userIdx, itemIdx, uEmbd, iEmbd, main_rows, main_cols, main_vals, trust_rows, trust_cols, trust_vals, add_rows, add_cols, add_vals, gnn_params, W1, b1, W2, b2, W3, b3):
    raise NotImplementedError("write your pallas kernel here")


# speedup vs baseline: 5.6519x; 5.6519x over previous
import functools

import jax
import jax.numpy as jnp
from jax import lax
from jax.experimental import pallas as pl
from jax.experimental.pallas import tpu as pltpu
from jax.experimental.pallas import tpu_sc as plsc

USER_NUM = 5000
N_NODES = 10000
D = 128
NNZ = 160000
NC = 2
NS = 16
EDGES_PER_TEC = NNZ // (NC * NS)
CHUNK = 128
N_FULL = EDGES_PER_TEC // CHUNK
TAIL = EDGES_PER_TEC - N_FULL * CHUNK
ACC_ROWS = 10112
ROWS_PER_TEC = ACC_ROWS // NS


def _spmm_body(p_hbm, rows_hbm, cols_hbm, vals_hbm, out_hbm,
               acc, g0, gt, cols0, rows0, vals0, colst, rowst, valst, sem):
    c = lax.axis_index("c")
    s = lax.axis_index("s")

    zero16 = jnp.zeros((16,), jnp.float32)

    def _zrow(k, _):
        for j in range(D // 16):
            g0[k, pl.ds(j * 16, 16)] = zero16
        return ()

    lax.fori_loop(0, CHUNK, _zrow, ())
    for i, (off, nrows) in enumerate([(0, 128), (128, 128), (256, 128),
                                      (384, 128), (512, 120)]):
        pltpu.sync_copy(g0.at[pl.ds(0, nrows)],
                        acc.at[pl.ds(s * ROWS_PER_TEC + off, nrows)])
    plsc.subcore_barrier()

    base0 = c * (NNZ // NC) + s * EDGES_PER_TEC

    def _chunk(base, n, gbuf, cbuf, rbuf, vbuf):
        pltpu.sync_copy(cols_hbm.at[pl.ds(base, n)], cbuf)
        pltpu.async_copy(p_hbm.at[cbuf], gbuf, sem).wait()
        pltpu.sync_copy(vals_hbm.at[pl.ds(base, n)],
                        vbuf if n >= 16 else vbuf.at[pl.ds(0, n)])
        pltpu.sync_copy(rows_hbm.at[pl.ds(base, n)], rbuf)

        def _scale_grp(g, _):
            vv = vbuf[pl.ds(g * 16, 16)]
            for i in range(16 if n >= 16 else n):
                v = vv[i]
                k = g * 16 + i
                for j in range(D // 16):
                    sl = pl.ds(j * 16, 16)
                    gbuf[k, sl] = gbuf[k, sl] * v
            return ()

        if n >= 16:
            lax.fori_loop(0, n // 16, _scale_grp, ())
        else:
            _scale_grp(0, ())
        pltpu.sync_copy(gbuf, acc.at[rbuf], add=True)

    def _full(i, _):
        _chunk(base0 + i * CHUNK, CHUNK, g0, cols0, rows0, vals0)
        return ()

    lax.fori_loop(0, N_FULL, _full, ())
    if TAIL:
        _chunk(base0 + N_FULL * CHUNK, TAIL, gt, colst, rowst, valst)
    plsc.subcore_barrier()

    pltpu.sync_copy(acc.at[pl.ds(s * ROWS_PER_TEC, ROWS_PER_TEC)],
                    out_hbm.at[c, pl.ds(s * ROWS_PER_TEC, ROWS_PER_TEC)])


_spmm_call = functools.partial(
    pl.kernel,
    out_type=jax.ShapeDtypeStruct((NC, ACC_ROWS, D), jnp.float32),
    mesh=plsc.VectorSubcoreMesh(core_axis_name="c", subcore_axis_name="s"),
    scratch_types=[
        pltpu.VMEM_SHARED((ACC_ROWS, D), jnp.float32),
        pltpu.VMEM((CHUNK, D), jnp.float32),
        pltpu.VMEM((TAIL, D), jnp.float32),
        pltpu.VMEM((CHUNK,), jnp.int32),
        pltpu.VMEM((CHUNK,), jnp.int32),
        pltpu.VMEM((CHUNK,), jnp.float32),
        pltpu.VMEM((TAIL,), jnp.int32),
        pltpu.VMEM((TAIL,), jnp.int32),
        pltpu.VMEM((16,), jnp.float32),
        pltpu.SemaphoreType.DMA,
    ],
)(_spmm_body)


def _spmm_sc(rows, cols, vals, p):
    parts = _spmm_call(p, rows, cols, vals)
    return parts[0, :N_NODES] + parts[1, :N_NODES]


def kernel(userIdx, itemIdx, uEmbd, iEmbd, main_rows, main_cols, main_vals,
           trust_rows, trust_cols, trust_vals, add_rows, add_cols, add_vals,
           gnn_params, W1, b1, W2, b2, W3, b3):
    feats = jnp.concatenate([uEmbd, iEmbd], axis=0)
    mats = [(main_rows.astype(jnp.int32), main_cols.astype(jnp.int32), main_vals),
            (trust_rows.astype(jnp.int32), trust_cols.astype(jnp.int32), trust_vals),
            (add_rows.astype(jnp.int32), add_cols.astype(jnp.int32), add_vals)]

    emb_sum = feats
    for p in gnn_params:
        f_lin = feats @ p['W_lin'].T
        pmat = f_lin + (feats * feats) @ p['W_inter'].T
        f0 = f_lin + p['b_lin'] + p['b_inter']
        ys, ws = [], []
        for name, (r, c, v) in zip(['main', 'trust', 'add'], mats):
            y = _spmm_sc(r, c, v, pmat) + f0
            ys.append(y)
            att = jnp.tanh(y @ p['W_att_' + name].T + p['b_att_' + name]) @ p['a_' + name]
            ws.append(att.mean())
        beta = jax.nn.softmax(jnp.stack([ws[0], ws[2], ws[1]]))
        feats = beta[0] * ys[0] + beta[1] * ys[2] + beta[2] * ys[1]
        emb_sum = emb_sum + feats

    final = emb_sum / 3.0
    userEmbd = final[userIdx]
    itemEmbd = final[itemIdx + USER_NUM]
    embd = jnp.concatenate([userEmbd, itemEmbd], axis=1)
    embd = jax.nn.relu(embd @ W1.T + b1)
    embd = embd @ W2.T + b2
    embd = embd @ W3.T + b3
    prediction = embd.reshape(-1)
    return prediction, userEmbd, itemEmbd, final


# speedup vs baseline: 5.7491x; 1.0172x over previous
import functools

import jax
import jax.numpy as jnp
from jax import lax
from jax.experimental import pallas as pl
from jax.experimental.pallas import tpu as pltpu
from jax.experimental.pallas import tpu_sc as plsc

USER_NUM = 5000
N_NODES = 10000
D = 128
NNZ = 160000
B = 4096
NC = 2
NS = 16
EDGES_PER_TEC = NNZ // (NC * NS)
CHUNK = 128
N_FULL = EDGES_PER_TEC // CHUNK
TAIL = EDGES_PER_TEC - N_FULL * CHUNK
ACC_ROWS = 10112
ROWS_PER_TEC = ACC_ROWS // NS
RB = 1000
NRB = N_NODES // RB
BB = 512
PAIRS_PER_TEC = B // (NC * NS)


def _spmm_body(p_hbm, rows_hbm, cols_hbm, vals_hbm, out_hbm,
               acc, g0, gt, cols0, rows0, vals0, colst, rowst, valst, sem):
    c = lax.axis_index("c")
    s = lax.axis_index("s")

    zero16 = jnp.zeros((16,), jnp.float32)

    def _zrow(k, _):
        for j in range(D // 16):
            g0[k, pl.ds(j * 16, 16)] = zero16
        return ()

    lax.fori_loop(0, CHUNK, _zrow, ())
    for off, nrows in [(0, 128), (128, 128), (256, 128), (384, 128), (512, 120)]:
        pltpu.sync_copy(g0.at[pl.ds(0, nrows)],
                        acc.at[pl.ds(s * ROWS_PER_TEC + off, nrows)])
    plsc.subcore_barrier()

    base0 = c * (NNZ // NC) + s * EDGES_PER_TEC

    def _chunk(base, n, gbuf, cbuf, rbuf, vbuf):
        pltpu.sync_copy(cols_hbm.at[pl.ds(base, n)], cbuf)
        pltpu.async_copy(p_hbm.at[cbuf], gbuf, sem).wait()
        pltpu.sync_copy(vals_hbm.at[pl.ds(base, n)],
                        vbuf if n >= 16 else vbuf.at[pl.ds(0, n)])
        pltpu.sync_copy(rows_hbm.at[pl.ds(base, n)], rbuf)

        def _scale_grp(g, _):
            vv = vbuf[pl.ds(g * 16, 16)]
            for i in range(16 if n >= 16 else n):
                v = vv[i]
                k = g * 16 + i
                for j in range(D // 16):
                    sl = pl.ds(j * 16, 16)
                    gbuf[k, sl] = gbuf[k, sl] * v
            return ()

        if n >= 16:
            lax.fori_loop(0, n // 16, _scale_grp, ())
        else:
            _scale_grp(0, ())
        pltpu.sync_copy(gbuf, acc.at[rbuf], add=True)

    def _full(i, _):
        _chunk(base0 + i * CHUNK, CHUNK, g0, cols0, rows0, vals0)
        return ()

    lax.fori_loop(0, N_FULL, _full, ())
    if TAIL:
        _chunk(base0 + N_FULL * CHUNK, TAIL, gt, colst, rowst, valst)
    plsc.subcore_barrier()

    pltpu.sync_copy(acc.at[pl.ds(s * ROWS_PER_TEC, ROWS_PER_TEC)],
                    out_hbm.at[c, pl.ds(s * ROWS_PER_TEC, ROWS_PER_TEC)])


_spmm_call = functools.partial(
    pl.kernel,
    out_type=jax.ShapeDtypeStruct((NC, ACC_ROWS, D), jnp.float32),
    mesh=plsc.VectorSubcoreMesh(core_axis_name="c", subcore_axis_name="s", num_cores=NC, num_subcores=NS),
    scratch_types=[
        pltpu.VMEM_SHARED((ACC_ROWS, D), jnp.float32),
        pltpu.VMEM((CHUNK, D), jnp.float32),
        pltpu.VMEM((TAIL, D), jnp.float32),
        pltpu.VMEM((CHUNK,), jnp.int32),
        pltpu.VMEM((CHUNK,), jnp.int32),
        pltpu.VMEM((CHUNK,), jnp.float32),
        pltpu.VMEM((TAIL,), jnp.int32),
        pltpu.VMEM((TAIL,), jnp.int32),
        pltpu.VMEM((16,), jnp.float32),
        pltpu.SemaphoreType.DMA,
    ],
)(_spmm_body)


def _gather_body(fin_hbm, uidx_hbm, iidx_hbm, uout_hbm, iout_hbm, ib, gb, sem):
    c = lax.axis_index("c")
    s = lax.axis_index("s")
    base = (s * NC + c) * PAIRS_PER_TEC
    pltpu.sync_copy(uidx_hbm.at[pl.ds(base, PAIRS_PER_TEC)], ib)
    pltpu.async_copy(fin_hbm.at[ib], gb, sem).wait()
    pltpu.sync_copy(gb, uout_hbm.at[pl.ds(base, PAIRS_PER_TEC)])
    pltpu.sync_copy(iidx_hbm.at[pl.ds(base, PAIRS_PER_TEC)], ib)
    pltpu.async_copy(fin_hbm.at[ib], gb, sem).wait()
    pltpu.sync_copy(gb, iout_hbm.at[pl.ds(base, PAIRS_PER_TEC)])


_gather_call = functools.partial(
    pl.kernel,
    out_type=(jax.ShapeDtypeStruct((B, D), jnp.float32),
              jax.ShapeDtypeStruct((B, D), jnp.float32)),
    mesh=plsc.VectorSubcoreMesh(core_axis_name="c", subcore_axis_name="s", num_cores=NC, num_subcores=NS),
    scratch_types=[
        pltpu.VMEM((PAIRS_PER_TEC,), jnp.int32),
        pltpu.VMEM((PAIRS_PER_TEC, D), jnp.float32),
        pltpu.SemaphoreType.DMA,
    ],
)(_gather_body)


_DN = (((1,), (1,)), ((), ()))


def _k1_body(f_ref, wl_ref, wi_ref, bl_ref, bi_ref, p_ref, f0_ref):
    f = f_ref[...]
    flin = jax.lax.dot_general(f, wl_ref[...], _DN,
                               preferred_element_type=jnp.float32)
    p_ref[...] = flin + jax.lax.dot_general(
        f * f, wi_ref[...], _DN, preferred_element_type=jnp.float32)
    f0_ref[...] = flin + bl_ref[...] + bi_ref[...]


def _k1(feats, wl, wi, bl, bi):
    return pl.pallas_call(
        _k1_body,
        name='k1_pretransform',
        grid=(NRB,),
        in_specs=[
            pl.BlockSpec((RB, D), lambda i: (i, 0)),
            pl.BlockSpec((D, D), lambda i: (0, 0)),
            pl.BlockSpec((D, D), lambda i: (0, 0)),
            pl.BlockSpec((D,), lambda i: (0,)),
            pl.BlockSpec((D,), lambda i: (0,)),
        ],
        out_specs=[
            pl.BlockSpec((RB, D), lambda i: (i, 0)),
            pl.BlockSpec((RB, D), lambda i: (i, 0)),
        ],
        out_shape=[
            jax.ShapeDtypeStruct((N_NODES, D), jnp.float32),
            jax.ShapeDtypeStruct((N_NODES, D), jnp.float32),
        ],
    )(feats, wl, wi, bl, bi)


def _k2_body(s_ref, f0_ref, wa_ref, ba_ref, y_ref, t_ref):
    y = s_ref[0] + s_ref[1] + f0_ref[...]
    y_ref[...] = y
    t = jnp.tanh(jax.lax.dot_general(y, wa_ref[...], _DN,
                                     preferred_element_type=jnp.float32)
                 + ba_ref[...])
    t_ref[...] = jnp.sum(t, axis=0).reshape(1, 1, D)


def _k2(s_parts, f0, wa, ba):
    return pl.pallas_call(
        _k2_body,
        name='k2_branch',
        grid=(NRB,),
        in_specs=[
            pl.BlockSpec((2, RB, D), lambda i: (0, i, 0)),
            pl.BlockSpec((RB, D), lambda i: (i, 0)),
            pl.BlockSpec((D, D), lambda i: (0, 0)),
            pl.BlockSpec((D,), lambda i: (0,)),
        ],
        out_specs=[
            pl.BlockSpec((RB, D), lambda i: (i, 0)),
            pl.BlockSpec((1, 1, D), lambda i: (i, 0, 0)),
        ],
        out_shape=[
            jax.ShapeDtypeStruct((N_NODES, D), jnp.float32),
            jax.ShapeDtypeStruct((NRB, 1, D), jnp.float32),
        ],
    )(s_parts, f0, wa, ba)


def _k3_body_l1(ym_ref, yt_ref, ya_ref, es_ref, beta_ref, f_ref, es_out_ref):
    f = (beta_ref[0] * ym_ref[...] + beta_ref[1] * ya_ref[...]
         + beta_ref[2] * yt_ref[...])
    f_ref[...] = f
    es_out_ref[...] = es_ref[...] + f


def _k3_body_l2(ym_ref, yt_ref, ya_ref, es_ref, beta_ref, f_ref, es_out_ref,
                fin_ref):
    f = (beta_ref[0] * ym_ref[...] + beta_ref[1] * ya_ref[...]
         + beta_ref[2] * yt_ref[...])
    f_ref[...] = f
    es = es_ref[...] + f
    es_out_ref[...] = es
    fin_ref[...] = es * (1.0 / 3.0)


def _k3(ym, yt, ya, es, beta, last):
    n_out = 3 if last else 2
    return pl.pallas_call(
        _k3_body_l2 if last else _k3_body_l1,
        name='k3_combine',
        grid=(NRB,),
        in_specs=[
            pl.BlockSpec((RB, D), lambda i: (i, 0)),
            pl.BlockSpec((RB, D), lambda i: (i, 0)),
            pl.BlockSpec((RB, D), lambda i: (i, 0)),
            pl.BlockSpec((RB, D), lambda i: (i, 0)),
            pl.BlockSpec(memory_space=pltpu.SMEM),
        ],
        out_specs=[pl.BlockSpec((RB, D), lambda i: (i, 0))] * n_out,
        out_shape=[jax.ShapeDtypeStruct((N_NODES, D), jnp.float32)] * n_out,
    )(ym, yt, ya, es, beta)


def _k4_body(u_ref, i_ref, w1u_ref, w1i_ref, b1_ref, w23_ref, b23_ref, p_ref):
    h = (jax.lax.dot_general(u_ref[...], w1u_ref[...], _DN,
                             preferred_element_type=jnp.float32)
         + jax.lax.dot_general(i_ref[...], w1i_ref[...], _DN,
                               preferred_element_type=jnp.float32)
         + b1_ref[...])
    h = jnp.maximum(h, 0.0)
    p_ref[...] = jnp.sum(h * w23_ref[...], axis=1, keepdims=True) + b23_ref[...]


def _k4(ue, ie, w1u, w1i, b1, w23, b23):
    return pl.pallas_call(
        _k4_body,
        name='k4_mlp',
        grid=(B // BB,),
        in_specs=[
            pl.BlockSpec((BB, D), lambda i: (i, 0)),
            pl.BlockSpec((BB, D), lambda i: (i, 0)),
            pl.BlockSpec((64, D), lambda i: (0, 0)),
            pl.BlockSpec((64, D), lambda i: (0, 0)),
            pl.BlockSpec((1, 64), lambda i: (0, 0)),
            pl.BlockSpec((1, 64), lambda i: (0, 0)),
            pl.BlockSpec((1, 1), lambda i: (0, 0)),
        ],
        out_specs=pl.BlockSpec((BB, 1), lambda i: (i, 0)),
        out_shape=jax.ShapeDtypeStruct((B, 1), jnp.float32),
    )(ue, ie, w1u, w1i, b1.reshape(1, 64), w23.reshape(1, 64), b23.reshape(1, 1))


def kernel(userIdx, itemIdx, uEmbd, iEmbd, main_rows, main_cols, main_vals,
           trust_rows, trust_cols, trust_vals, add_rows, add_cols, add_vals,
           gnn_params, W1, b1, W2, b2, W3, b3):
    feats = jnp.concatenate([uEmbd, iEmbd], axis=0)
    mats = [(main_rows.astype(jnp.int32), main_cols.astype(jnp.int32), main_vals),
            (trust_rows.astype(jnp.int32), trust_cols.astype(jnp.int32), trust_vals),
            (add_rows.astype(jnp.int32), add_cols.astype(jnp.int32), add_vals)]

    emb_sum = feats
    final = None
    for li, p in enumerate(gnn_params):
        pmat, f0 = _k1(feats, p['W_lin'], p['W_inter'], p['b_lin'], p['b_inter'])
        ys, ws = [], []
        for name, (r, c, v) in zip(['main', 'trust', 'add'], mats):
            s_parts = _spmm_call(pmat, r, c, v)
            y, tsum = _k2(s_parts, f0, p['W_att_' + name], p['b_att_' + name])
            ys.append(y)
            ws.append((tsum.sum((0, 1)) @ p['a_' + name])[0] * (1.0 / N_NODES))
        beta = jax.nn.softmax(jnp.stack([ws[0], ws[2], ws[1]]))
        last = li == len(gnn_params) - 1
        outs = _k3(ys[0], ys[1], ys[2], emb_sum, beta, last)
        if last:
            feats, emb_sum, final = outs
        else:
            feats, emb_sum = outs

    userEmbd, itemEmbd = _gather_call(
        final, userIdx.astype(jnp.int32),
        (itemIdx + USER_NUM).astype(jnp.int32))
    w23 = (W3 @ W2).reshape(-1)
    b23 = W3 @ b2 + b3
    prediction = _k4(userEmbd, itemEmbd, W1[:, :D], W1[:, D:], b1,
                     w23, b23).reshape(-1)
    return prediction, userEmbd, itemEmbd, final


# speedup vs baseline: 8.9003x; 1.5481x over previous
import functools

import jax
import jax.numpy as jnp
from jax import lax
from jax.experimental import pallas as pl
from jax.experimental.pallas import tpu as pltpu
from jax.experimental.pallas import tpu_sc as plsc

USER_NUM = 5000
N_NODES = 10000
D = 128
NNZ = 160000
B = 4096
NC = 2
NS = 16
NW = NC * NS
CHUNK = 128
NCHUNKS = NNZ // CHUNK
NCHUNK_BASE = NCHUNKS // NW
NXTRA = NCHUNKS - NCHUNK_BASE * NW
ACC_ROWS = 10112
ROWS_PER_TEC = ACC_ROWS // NS
RB = 1000
NRB = N_NODES // RB
BB = 512
PAIRS_PER_TEC = B // (NC * NS)


def _spmm_body(p_hbm, rows_hbm, cols_hbm, vals_hbm, out_hbm,
               acc, gA, gB, colsb, rowsb, valsb,
               semA, sgA, sgB, ssA, ssB):
    c = lax.axis_index("c")
    s = lax.axis_index("s")
    w = s * NC + c

    def _one_matrix(m, _):
        descs = []
        for i in range(NCHUNK_BASE):
            off = (w * NCHUNK_BASE + i) * CHUNK
            descs.append(pltpu.async_copy(rows_hbm.at[pl.ds(m * NNZ + off, CHUNK)], rowsb.at[i], semA))
            descs.append(pltpu.async_copy(cols_hbm.at[pl.ds(m * NNZ + off, CHUNK)], colsb.at[i], semA))
            descs.append(pltpu.async_copy(vals_hbm.at[pl.ds(m * NNZ + off, CHUNK)], valsb.at[i], semA))

        @pl.when(w < NXTRA)
        def _stage_extra():
            xoff = (NW * NCHUNK_BASE + w) * CHUNK
            pltpu.async_copy(rows_hbm.at[pl.ds(m * NNZ + xoff, CHUNK)], rowsb.at[NCHUNK_BASE], semA)
            pltpu.async_copy(cols_hbm.at[pl.ds(m * NNZ + xoff, CHUNK)], colsb.at[NCHUNK_BASE], semA)
            pltpu.async_copy(vals_hbm.at[pl.ds(m * NNZ + xoff, CHUNK)], valsb.at[NCHUNK_BASE], semA)

        zero16 = jnp.zeros((16,), jnp.float32)

        def _zrow(k, _):
            for j in range(D // 16):
                gA[k, pl.ds(j * 16, 16)] = zero16
            return ()

        lax.fori_loop(0, CHUNK, _zrow, ())
        for off, nrows in [(0, 128), (128, 128), (256, 128), (384, 128), (512, 120)]:
            pltpu.sync_copy(gA.at[pl.ds(0, nrows)],
                            acc.at[pl.ds(s * ROWS_PER_TEC + off, nrows)])

        @pl.when(w < NXTRA)
        def _drain_extra():
            xoff = (NW * NCHUNK_BASE + w) * CHUNK
            pltpu.make_async_copy(rows_hbm.at[pl.ds(m * NNZ + xoff, CHUNK)], rowsb.at[NCHUNK_BASE], semA).wait()
            pltpu.make_async_copy(cols_hbm.at[pl.ds(m * NNZ + xoff, CHUNK)], colsb.at[NCHUNK_BASE], semA).wait()
            pltpu.make_async_copy(vals_hbm.at[pl.ds(m * NNZ + xoff, CHUNK)], valsb.at[NCHUNK_BASE], semA).wait()

        for d in descs:
            d.wait()
        plsc.subcore_barrier()

        nch = jnp.where(w < NXTRA, NCHUNK_BASE + 1, NCHUNK_BASE)

        def _scale(gbuf, i):
            def _grp(g, _):
                vv = valsb[i, pl.ds(g * 16, 16)]
                for u in range(16):
                    v = vv[u]
                    k = g * 16 + u
                    for j in range(D // 16):
                        sl = pl.ds(j * 16, 16)
                        gbuf[k, sl] = gbuf[k, sl] * v
                return ()

            lax.fori_loop(0, CHUNK // 16, _grp, ())

        def _phase(i, cur, sg_cur, ss_cur, oth, sg_oth, ss_oth):
            pltpu.make_async_copy(p_hbm.at[colsb.at[i]], cur, sg_cur).wait()
            _scale(cur, i)

            @pl.when(i >= 1)
            def _drain_oth():
                pltpu.make_async_copy(oth, acc.at[rowsb.at[0]], ss_oth).wait()

            @pl.when(i + 1 < nch)
            def _issue_next():
                pltpu.async_copy(p_hbm.at[colsb.at[i + 1]], oth, sg_oth)

            pltpu.async_copy(cur, acc.at[rowsb.at[i]], ss_cur, add=True)

        def _pair(p, _):
            i0 = 2 * p
            _phase(i0, gA, sgA, ssA, gB, sgB, ssB)
            _phase(i0 + 1, gB, sgB, ssB, gA, sgA, ssA)
            return ()

        pltpu.async_copy(p_hbm.at[colsb.at[0]], gA, sgA)
        lax.fori_loop(0, NCHUNK_BASE // 2, _pair, ())

        @pl.when(w < NXTRA)
        def _last_pair():
            _pair(NCHUNK_BASE // 2, ())
            pltpu.make_async_copy(gB, acc.at[rowsb.at[0]], ssB).wait()

        @pl.when(w >= NXTRA)
        def _last_single():
            _phase(NCHUNK_BASE - 1, gA, sgA, ssA, gB, sgB, ssB)
            pltpu.make_async_copy(gA, acc.at[rowsb.at[0]], ssA).wait()

        plsc.subcore_barrier()

        pltpu.sync_copy(acc.at[pl.ds(s * ROWS_PER_TEC, ROWS_PER_TEC)],
                        out_hbm.at[m, c, pl.ds(s * ROWS_PER_TEC, ROWS_PER_TEC)])
        plsc.subcore_barrier()
        return ()

    lax.fori_loop(0, 3, _one_matrix, ())


_spmm_call = functools.partial(
    pl.kernel,
    out_type=jax.ShapeDtypeStruct((3, NC, ACC_ROWS, D), jnp.float32),
    mesh=plsc.VectorSubcoreMesh(core_axis_name="c", subcore_axis_name="s",
                                num_cores=NC, num_subcores=NS),
    scratch_types=[
        pltpu.VMEM_SHARED((ACC_ROWS, D), jnp.float32),
        pltpu.VMEM((CHUNK, D), jnp.float32),
        pltpu.VMEM((CHUNK, D), jnp.float32),
        pltpu.VMEM((NCHUNK_BASE + 1, CHUNK), jnp.int32),
        pltpu.VMEM((NCHUNK_BASE + 1, CHUNK), jnp.int32),
        pltpu.VMEM((NCHUNK_BASE + 1, CHUNK), jnp.float32),
        pltpu.SemaphoreType.DMA,
        pltpu.SemaphoreType.DMA,
        pltpu.SemaphoreType.DMA,
        pltpu.SemaphoreType.DMA,
        pltpu.SemaphoreType.DMA,
    ],
)(_spmm_body)


def _gather_body(fin_hbm, uidx_hbm, iidx_hbm, uout_hbm, iout_hbm, ib, gb, sem):
    c = lax.axis_index("c")
    s = lax.axis_index("s")
    base = (s * NC + c) * PAIRS_PER_TEC
    pltpu.sync_copy(uidx_hbm.at[pl.ds(base, PAIRS_PER_TEC)], ib)
    pltpu.async_copy(fin_hbm.at[ib], gb, sem).wait()
    pltpu.sync_copy(gb, uout_hbm.at[pl.ds(base, PAIRS_PER_TEC)])
    pltpu.sync_copy(iidx_hbm.at[pl.ds(base, PAIRS_PER_TEC)], ib)
    pltpu.async_copy(fin_hbm.at[ib], gb, sem).wait()
    pltpu.sync_copy(gb, iout_hbm.at[pl.ds(base, PAIRS_PER_TEC)])


_gather_call = functools.partial(
    pl.kernel,
    out_type=(jax.ShapeDtypeStruct((B, D), jnp.float32),
              jax.ShapeDtypeStruct((B, D), jnp.float32)),
    mesh=plsc.VectorSubcoreMesh(core_axis_name="c", subcore_axis_name="s", num_cores=NC, num_subcores=NS),
    scratch_types=[
        pltpu.VMEM((PAIRS_PER_TEC,), jnp.int32),
        pltpu.VMEM((PAIRS_PER_TEC, D), jnp.float32),
        pltpu.SemaphoreType.DMA,
    ],
)(_gather_body)


_DN = (((1,), (1,)), ((), ()))


def _k1_body(f_ref, wl_ref, wi_ref, bl_ref, bi_ref, p_ref, f0_ref):
    f = f_ref[...]
    flin = jax.lax.dot_general(f, wl_ref[...], _DN,
                               preferred_element_type=jnp.float32)
    p_ref[...] = flin + jax.lax.dot_general(
        f * f, wi_ref[...], _DN, preferred_element_type=jnp.float32)
    f0_ref[...] = flin + bl_ref[...] + bi_ref[...]


def _k1(feats, wl, wi, bl, bi):
    return pl.pallas_call(
        _k1_body,
        name='k1_pretransform',
        grid=(NRB,),
        in_specs=[
            pl.BlockSpec((RB, D), lambda i: (i, 0)),
            pl.BlockSpec((D, D), lambda i: (0, 0)),
            pl.BlockSpec((D, D), lambda i: (0, 0)),
            pl.BlockSpec((D,), lambda i: (0,)),
            pl.BlockSpec((D,), lambda i: (0,)),
        ],
        out_specs=[
            pl.BlockSpec((RB, D), lambda i: (i, 0)),
            pl.BlockSpec((RB, D), lambda i: (i, 0)),
        ],
        out_shape=[
            jax.ShapeDtypeStruct((N_NODES, D), jnp.float32),
            jax.ShapeDtypeStruct((N_NODES, D), jnp.float32),
        ],
    )(feats, wl, wi, bl, bi)


def _k2_body(s_ref, f0_ref, wa_ref, ba_ref, y_ref, t_ref):
    y = s_ref[0, 0] + s_ref[0, 1] + f0_ref[...]
    y_ref[...] = y
    t = jnp.tanh(jax.lax.dot_general(y, wa_ref[...], _DN,
                                     preferred_element_type=jnp.float32)
                 + ba_ref[...])
    t_ref[...] = jnp.sum(t, axis=0).reshape(1, 1, D)


def _k2(s3, mi, f0, wa, ba):
    return pl.pallas_call(
        _k2_body,
        name='k2_branch',
        grid=(NRB,),
        in_specs=[
            pl.BlockSpec((1, 2, RB, D), lambda i: (mi, 0, i, 0)),
            pl.BlockSpec((RB, D), lambda i: (i, 0)),
            pl.BlockSpec((D, D), lambda i: (0, 0)),
            pl.BlockSpec((D,), lambda i: (0,)),
        ],
        out_specs=[
            pl.BlockSpec((RB, D), lambda i: (i, 0)),
            pl.BlockSpec((1, 1, D), lambda i: (i, 0, 0)),
        ],
        out_shape=[
            jax.ShapeDtypeStruct((N_NODES, D), jnp.float32),
            jax.ShapeDtypeStruct((NRB, 1, D), jnp.float32),
        ],
    )(s3, f0, wa, ba)


def _k3_body_l1(ym_ref, yt_ref, ya_ref, es_ref, beta_ref, f_ref, es_out_ref):
    f = (beta_ref[0] * ym_ref[...] + beta_ref[1] * ya_ref[...]
         + beta_ref[2] * yt_ref[...])
    f_ref[...] = f
    es_out_ref[...] = es_ref[...] + f


def _k3_body_l2(ym_ref, yt_ref, ya_ref, es_ref, beta_ref, f_ref, es_out_ref,
                fin_ref):
    f = (beta_ref[0] * ym_ref[...] + beta_ref[1] * ya_ref[...]
         + beta_ref[2] * yt_ref[...])
    f_ref[...] = f
    es = es_ref[...] + f
    es_out_ref[...] = es
    fin_ref[...] = es * (1.0 / 3.0)


def _k3(ym, yt, ya, es, beta, last):
    n_out = 3 if last else 2
    return pl.pallas_call(
        _k3_body_l2 if last else _k3_body_l1,
        name='k3_combine',
        grid=(NRB,),
        in_specs=[
            pl.BlockSpec((RB, D), lambda i: (i, 0)),
            pl.BlockSpec((RB, D), lambda i: (i, 0)),
            pl.BlockSpec((RB, D), lambda i: (i, 0)),
            pl.BlockSpec((RB, D), lambda i: (i, 0)),
            pl.BlockSpec(memory_space=pltpu.SMEM),
        ],
        out_specs=[pl.BlockSpec((RB, D), lambda i: (i, 0))] * n_out,
        out_shape=[jax.ShapeDtypeStruct((N_NODES, D), jnp.float32)] * n_out,
    )(ym, yt, ya, es, beta)


def _k4_body(u_ref, i_ref, w1u_ref, w1i_ref, b1_ref, w23_ref, b23_ref, p_ref):
    h = (jax.lax.dot_general(u_ref[...], w1u_ref[...], _DN,
                             preferred_element_type=jnp.float32)
         + jax.lax.dot_general(i_ref[...], w1i_ref[...], _DN,
                               preferred_element_type=jnp.float32)
         + b1_ref[...])
    h = jnp.maximum(h, 0.0)
    p_ref[...] = jnp.sum(h * w23_ref[...], axis=1, keepdims=True) + b23_ref[...]


def _k4(ue, ie, w1u, w1i, b1, w23, b23):
    return pl.pallas_call(
        _k4_body,
        name='k4_mlp',
        grid=(B // BB,),
        in_specs=[
            pl.BlockSpec((BB, D), lambda i: (i, 0)),
            pl.BlockSpec((BB, D), lambda i: (i, 0)),
            pl.BlockSpec((64, D), lambda i: (0, 0)),
            pl.BlockSpec((64, D), lambda i: (0, 0)),
            pl.BlockSpec((1, 64), lambda i: (0, 0)),
            pl.BlockSpec((1, 64), lambda i: (0, 0)),
            pl.BlockSpec((1, 1), lambda i: (0, 0)),
        ],
        out_specs=pl.BlockSpec((BB, 1), lambda i: (i, 0)),
        out_shape=jax.ShapeDtypeStruct((B, 1), jnp.float32),
    )(ue, ie, w1u, w1i, b1.reshape(1, 64), w23.reshape(1, 64), b23.reshape(1, 1))


def kernel(userIdx, itemIdx, uEmbd, iEmbd, main_rows, main_cols, main_vals,
           trust_rows, trust_cols, trust_vals, add_rows, add_cols, add_vals,
           gnn_params, W1, b1, W2, b2, W3, b3):
    feats = jnp.concatenate([uEmbd, iEmbd], axis=0)
    mats = [(main_rows.astype(jnp.int32), main_cols.astype(jnp.int32), main_vals),
            (trust_rows.astype(jnp.int32), trust_cols.astype(jnp.int32), trust_vals),
            (add_rows.astype(jnp.int32), add_cols.astype(jnp.int32), add_vals)]

    rows3 = jnp.concatenate([mats[0][0], mats[1][0], mats[2][0]])
    cols3 = jnp.concatenate([mats[0][1], mats[1][1], mats[2][1]])
    vals3 = jnp.concatenate([mats[0][2], mats[1][2], mats[2][2]])

    emb_sum = feats
    final = None
    for li, p in enumerate(gnn_params):
        pmat, f0 = _k1(feats, p['W_lin'], p['W_inter'], p['b_lin'], p['b_inter'])
        s3 = _spmm_call(pmat, rows3, cols3, vals3)
        ys, ws = [], []
        for mi, name in enumerate(['main', 'trust', 'add']):
            y, tsum = _k2(s3, mi, f0, p['W_att_' + name], p['b_att_' + name])
            ys.append(y)
            ws.append((tsum.sum((0, 1)) @ p['a_' + name])[0] * (1.0 / N_NODES))
        beta = jax.nn.softmax(jnp.stack([ws[0], ws[2], ws[1]]))
        last = li == len(gnn_params) - 1
        outs = _k3(ys[0], ys[1], ys[2], emb_sum, beta, last)
        if last:
            feats, emb_sum, final = outs
        else:
            feats, emb_sum = outs

    userEmbd, itemEmbd = _gather_call(
        final, userIdx.astype(jnp.int32),
        (itemIdx + USER_NUM).astype(jnp.int32))
    w23 = (W3 @ W2).reshape(-1)
    b23 = W3 @ b2 + b3
    prediction = _k4(userEmbd, itemEmbd, W1[:, :D], W1[:, D:], b1,
                     w23, b23).reshape(-1)
    return prediction, userEmbd, itemEmbd, final
